# baseline (speedup 1.0000x reference)
import functools

import jax
import jax.numpy as jnp
import numpy as np
from jax import lax
from jax.experimental import pallas as pl
from jax.experimental.pallas import tpu as pltpu
from jax.experimental.pallas import tpu_sc as plsc

NC = 2
NS = 16
NW = NC * NS
NCH = 25
CW = 128
N = 100000
STRIDE = 3128
LAST = N - NCH * CW
_BASES = np.minimum(np.arange(NW) * STRIDE, LAST)


def _sc_gather(table, idx3, n):
    mesh = plsc.VectorSubcoreMesh(core_axis_name="c", subcore_axis_name="s")

    @functools.partial(
        pl.kernel,
        out_type=jax.ShapeDtypeStruct((n, 256), table.dtype),
        mesh=mesh,
        scratch_types=[
            pltpu.VMEM((NCH, CW), jnp.int32),
            pltpu.VMEM((CW, 256), jnp.float32),
            pltpu.SemaphoreType.DMA,
        ],
    )
    def run(table_hbm, idx_hbm, out_hbm, idx_v, buf_v, gsem):
        wid = lax.axis_index("s") * NC + lax.axis_index("c")
        base = lax.min(wid * STRIDE, LAST)
        pltpu.sync_copy(idx_hbm.at[wid], idx_v)

        def body(g, carry):
            pltpu.async_copy(table_hbm.at[idx_v.at[g]], buf_v, gsem).wait()
            pltpu.sync_copy(buf_v, out_hbm.at[pl.ds(base + g * CW, CW)])
            return carry

        lax.fori_loop(0, NCH, body, 0)

    return run(table, idx3)


def kernel(emb0, x):
    idx = x.astype(jnp.int32)[:, 0]
    take = jnp.asarray(_BASES[:, None] + np.arange(NCH * CW)[None, :])
    idx3 = idx[take].reshape(NW, NCH, CW)
    return _sc_gather(emb0, idx3, N)


# speedup vs baseline: 1.0313x; 1.0313x over previous
import functools

import jax
import jax.numpy as jnp
from jax import lax
from jax.experimental import pallas as pl
from jax.experimental.pallas import tpu as pltpu
from jax.experimental.pallas import tpu_sc as plsc

NC = 2
NS = 16
NW = NC * NS
NCH = 25
CW = 128
NBUF = 3
N = 100000
D = 256
STRIDE = 3128
LAST = N - NCH * CW


def _sc_gather(table, idx):
    mesh = plsc.VectorSubcoreMesh(core_axis_name="c", subcore_axis_name="s")

    @functools.partial(
        pl.kernel,
        out_type=jax.ShapeDtypeStruct((N, D), table.dtype),
        mesh=mesh,
        scratch_types=[
            pltpu.VMEM((NCH * CW,), jnp.int32),
            pltpu.VMEM((NBUF, CW, D), jnp.float32),
            pltpu.SemaphoreType.DMA((NBUF,)),
            pltpu.SemaphoreType.DMA((NBUF,)),
        ],
    )
    def run(table_hbm, idx_hbm, out_hbm, idx_v, buf_v, gsem, ssem):
        wid = lax.axis_index("s") * NC + lax.axis_index("c")
        base = lax.min(wid * STRIDE, LAST)
        pltpu.sync_copy(idx_hbm.at[pl.ds(base, NCH * CW)], idx_v)

        def sg(g):
            b = g % NBUF
            return pltpu.async_copy(
                table_hbm.at[idx_v.at[pl.ds(g * CW, CW)]],
                buf_v.at[b], gsem.at[b])

        def ss(g):
            b = g % NBUF
            return pltpu.async_copy(
                buf_v.at[b], out_hbm.at[pl.ds(base + g * CW, CW)],
                ssem.at[b])

        gd = [None] * NCH
        sd = [None] * NCH
        gd[0] = sg(0)
        gd[1] = sg(1)
        gd[0].wait()
        sd[0] = ss(0)
        gd[2] = sg(2)
        gd[1].wait()
        sd[1] = ss(1)
        for g in range(2, NCH):
            if g + 1 < NCH:
                sd[g - 2].wait()
                gd[g + 1] = sg(g + 1)
            gd[g].wait()
            sd[g] = ss(g)
        sd[NCH - 3].wait()
        sd[NCH - 2].wait()
        sd[NCH - 1].wait()

    return run(table, idx)


def kernel(emb0, x):
    return _sc_gather(emb0, x.astype(jnp.int32).reshape(N))
